# baseline (speedup 1.0000x reference)
import functools

import jax
import jax.numpy as jnp
from jax import lax
from jax.experimental import pallas as pl
from jax.experimental.pallas import tpu as pltpu
from jax.experimental.pallas import tpu_sc as plsc

DIM = 32
BATCH = 16384
L = 16
NC = 2
NS = 16
NW = NC * NS
BPW = BATCH // NW
CHUNK = 128
NCH = BPW // CHUNK

_mesh = plsc.VectorSubcoreMesh(core_axis_name="c", subcore_axis_name="s")


@functools.partial(
    pl.kernel,
    out_type=jax.ShapeDtypeStruct((BATCH,), jnp.float32),
    mesh=_mesh,
    scratch_types=[
        pltpu.VMEM((3, NCH, CHUNK), jnp.int32),
        pltpu.VMEM((BPW, DIM), jnp.float32),
        pltpu.VMEM((BPW, DIM), jnp.float32),
        pltpu.VMEM((BPW, DIM), jnp.float32),
        pltpu.VMEM((BPW,), jnp.float32),
        pltpu.SemaphoreType.DMA,
    ],
    compiler_params=pltpu.CompilerParams(
        needs_layout_passes=False, use_tc_tiling_on_sc=False),
)
def _transe_sc(ent, rel, idx, out, idx_v, r1_v, rr_v, r2_v, out_v, sem):
    wid = lax.axis_index("s") * NC + lax.axis_index("c")
    base = wid * BPW

    pltpu.sync_copy(idx.at[wid], idx_v)

    copies = []
    for k in range(NCH):
        dst = pl.ds(k * CHUNK, CHUNK)
        copies.append(pltpu.async_copy(ent.at[idx_v.at[0, k]], r1_v.at[dst], sem))
        copies.append(pltpu.async_copy(rel.at[idx_v.at[1, k]], rr_v.at[dst], sem))
        copies.append(pltpu.async_copy(ent.at[idx_v.at[2, k]], r2_v.at[dst], sem))
    for c in copies:
        c.wait()

    cols = [jnp.full((L,), j, jnp.int32) for j in range(DIM)]

    def group(g, carry):
        rows = g * L + lax.iota(jnp.int32, L)
        n1 = jnp.zeros((L,), jnp.float32)
        nr = jnp.zeros((L,), jnp.float32)
        n2 = jnp.zeros((L,), jnp.float32)
        for j in range(DIM):
            n1 = n1 + jnp.abs(plsc.load_gather(r1_v, [rows, cols[j]]))
            nr = nr + jnp.abs(plsc.load_gather(rr_v, [rows, cols[j]]))
            n2 = n2 + jnp.abs(plsc.load_gather(r2_v, [rows, cols[j]]))
        s1 = 1.0 / n1
        sr = 1.0 / nr
        s2 = 1.0 / n2
        acc = jnp.zeros((L,), jnp.float32)
        for j in range(DIM):
            a = plsc.load_gather(r1_v, [rows, cols[j]])
            b = plsc.load_gather(rr_v, [rows, cols[j]])
            d = plsc.load_gather(r2_v, [rows, cols[j]])
            acc = acc + jnp.abs(a * s1 + b * sr - d * s2)
        out_v[pl.ds(g * L, L)] = acc
        return carry

    lax.fori_loop(0, BPW // L, group, 0)

    pltpu.sync_copy(out_v, out.at[pl.ds(base, BPW)])


@jax.jit
def kernel(batch_inputs, entity_weight, relation_weight):
    bi = batch_inputs.astype(jnp.int32)
    idx = bi.reshape(NW, NCH, CHUNK, 3).transpose(0, 3, 1, 2)
    return _transe_sc(entity_weight, relation_weight, idx)
